# baseline (speedup 1.0000x reference)
import jax
import jax.numpy as jnp
from jax import lax
from jax.experimental import pallas as pl
from jax.experimental.pallas import tpu as pltpu
from jax.experimental.pallas import tpu_sc as plsc

_D = _H = _W = 64
_EMB = 64
_ROW = 3 * _EMB
_NV = _EMB // 16
_HB = 4
_BR = _HB * _W
_TAB = _D * _EMB
_BLK = _BR * _ROW


def _body(d_hbm, h_hbm, w_hbm, out_hbm, tab_d, tab_h, tab_w, blk0, blk1,
          sem0, sem1):
    wid = lax.axis_index("s") * 2 + lax.axis_index("c")

    pltpu.sync_copy(d_hbm.at[pl.ds(0, _TAB)], tab_d)
    pltpu.sync_copy(h_hbm.at[pl.ds(0, _TAB)], tab_h)
    pltpu.sync_copy(w_hbm.at[pl.ds(0, _TAB)], tab_w)

    blks = (blk0, blk1)
    sems = (sem0, sem1)

    def fill_w(r, carry):
        for k in range(_NV):
            v = tab_w[pl.ds(r * _EMB + 16 * k, 16)]
            for j in range(_HB):
                off = (j * _W + r) * _ROW + 2 * _EMB + 16 * k
                blk0[pl.ds(off, 16)] = v
                blk1[pl.ds(off, 16)] = v
        return carry
    lax.fori_loop(0, _W, fill_w, 0)

    def fill_h(h0, blk):
        for j in range(_HB):
            hv = [tab_h[pl.ds((h0 + j) * _EMB + 16 * k, 16)]
                  for k in range(_NV)]
            def body(r, carry, j=j, hv=hv):
                for k in range(_NV):
                    blk[pl.ds((j * _W + r) * _ROW + _EMB + 16 * k, 16)] = \
                        hv[k]
                return carry
            lax.fori_loop(0, _W, body, 0)

    for dd in range(2):
        d = wid * 2 + dd
        dv = [tab_d[pl.ds(d * _EMB + 16 * k, 16)] for k in range(_NV)]

        def fill_d(r, carry):
            for k in range(_NV):
                blk0[pl.ds(r * _ROW + 16 * k, 16)] = dv[k]
                blk1[pl.ds(r * _ROW + 16 * k, 16)] = dv[k]
            return carry
        lax.fori_loop(0, _BR, fill_d, 0)

        base = d * (_H * _W) * _ROW
        nblk = _H // _HB

        for p in range(2):
            fill_h(p * _HB, blks[p])
            pltpu.async_copy(
                blks[p], out_hbm.at[pl.ds(base + p * _BLK, _BLK)], sems[p])

        def pipe(i, carry):
            for p in range(2):
                hc = 2 * i + p
                pltpu.make_async_copy(
                    blks[p], out_hbm.at[pl.ds(base, _BLK)], sems[p]).wait()
                fill_h(hc * _HB, blks[p])
                pltpu.async_copy(
                    blks[p], out_hbm.at[pl.ds(base + hc * _BLK, _BLK)],
                    sems[p])
            return carry
        lax.fori_loop(1, nblk // 2, pipe, 0)

        for p in range(2):
            pltpu.make_async_copy(
                blks[p], out_hbm.at[pl.ds(base, _BLK)], sems[p]).wait()


def kernel(depth, height, width, batch_size, d_emb, h_emb, w_emb):
    mesh = plsc.VectorSubcoreMesh(core_axis_name="c", subcore_axis_name="s")
    out = pl.kernel(
        _body,
        out_type=jax.ShapeDtypeStruct((_D * _H * _W * _ROW,), jnp.float32),
        mesh=mesh,
        scratch_types=[
            pltpu.VMEM((_TAB,), jnp.float32),
            pltpu.VMEM((_TAB,), jnp.float32),
            pltpu.VMEM((_TAB,), jnp.float32),
            pltpu.VMEM((_BLK,), jnp.float32),
            pltpu.VMEM((_BLK,), jnp.float32),
            pltpu.SemaphoreType.DMA,
            pltpu.SemaphoreType.DMA,
        ],
    )(d_emb.reshape(-1), h_emb.reshape(-1), w_emb.reshape(-1))
    return out.reshape(1, _D * _H * _W, _ROW)
